# Initial kernel scaffold; baseline (speedup 1.0000x reference)
import jax
import jax.numpy as jnp
from jax.experimental import pallas as pl


def kernel(logits, targets):
    raise NotImplementedError("write your pallas kernel here")


# speedup vs baseline: 6.8992x; 6.8992x over previous
import functools

import jax
import jax.numpy as jnp
from jax.experimental import pallas as pl
from jax.experimental.pallas import tpu as pltpu

_BH = 64


def _pce_block(logits_ref, targets_ref, nll_sum_ref, count_ref):
    step = pl.program_id(0) * pl.num_programs(1) + pl.program_id(1)

    @pl.when(step == 0)
    def _init():
        nll_sum_ref[0, 0] = 0.0
        count_ref[0, 0] = 0.0

    x = logits_ref[0]
    t = targets_ref[0]
    c = x.shape[0]

    valid = t != -1
    t_safe = jnp.where(valid, t, 0)

    m = jnp.max(x, axis=0)
    cls = jax.lax.broadcasted_iota(jnp.int32, x.shape, 0)
    e = jnp.sum(jnp.exp(x - m[None]), axis=0)
    picked = jnp.sum(jnp.where(cls == t_safe[None], x, 0.0), axis=0)

    nll = (m + jnp.log(e) - picked) * valid.astype(jnp.float32)
    nll_sum_ref[0, 0] += jnp.sum(nll)
    count_ref[0, 0] += jnp.sum(valid.astype(jnp.float32))


@jax.jit
def kernel(logits, targets):
    B, C, H, W = logits.shape
    grid = (B, H // _BH)
    nll_sum, count = pl.pallas_call(
        _pce_block,
        grid=grid,
        in_specs=[
            pl.BlockSpec((1, C, _BH, W), lambda b, j: (b, 0, j, 0)),
            pl.BlockSpec((1, _BH, W), lambda b, j: (b, j, 0)),
        ],
        out_specs=[
            pl.BlockSpec(memory_space=pltpu.SMEM, block_shape=(1, 1),
                         index_map=lambda b, j: (0, 0)),
            pl.BlockSpec(memory_space=pltpu.SMEM, block_shape=(1, 1),
                         index_map=lambda b, j: (0, 0)),
        ],
        out_shape=[
            jax.ShapeDtypeStruct((1, 1), jnp.float32),
            jax.ShapeDtypeStruct((1, 1), jnp.float32),
        ],
    )(logits, targets)
    nll_sum = nll_sum[0, 0]
    count = count[0, 0]
    loss = nll_sum / jnp.maximum(count, 1.0)
    return jnp.where(count == 0.0, jnp.float32(0.0), loss)
